# baseline (speedup 1.0000x reference)
import jax
import jax.numpy as jnp
from jax.experimental import pallas as pl
from jax.experimental.pallas import tpu as pltpu

NUM_PROMPTS = 50
PROMPT_DIM = 768
SEQ = 577


def _splice_kernel(layer_ref, x_ref, pe_ref, o_ref):
    del layer_ref
    o_ref[0, 0:1, :] = x_ref[0, 0:1, :]
    o_ref[0, 1:1 + NUM_PROMPTS, :] = pe_ref[0]
    o_ref[0, 1 + NUM_PROMPTS:, :] = x_ref[0, 1:, :]


def kernel(x, prompt_embeddings, layer_num):
    Bsz = x.shape[0]
    layer = jnp.asarray(layer_num, jnp.int32).reshape((1,))
    grid_spec = pltpu.PrefetchScalarGridSpec(
        num_scalar_prefetch=1,
        grid=(Bsz,),
        in_specs=[
            pl.BlockSpec((1, SEQ, PROMPT_DIM), lambda b, s: (b, 0, 0)),
            pl.BlockSpec((1, NUM_PROMPTS, PROMPT_DIM),
                         lambda b, s: (s[0], 0, 0)),
        ],
        out_specs=pl.BlockSpec((1, SEQ + NUM_PROMPTS, PROMPT_DIM),
                               lambda b, s: (b, 0, 0)),
    )
    return pl.pallas_call(
        _splice_kernel,
        grid_spec=grid_spec,
        out_shape=jax.ShapeDtypeStruct((Bsz, SEQ + NUM_PROMPTS, PROMPT_DIM),
                                       x.dtype),
    )(layer, x, prompt_embeddings)


# speedup vs baseline: 1.0408x; 1.0408x over previous
import jax
import jax.numpy as jnp
from jax.experimental import pallas as pl
from jax.experimental.pallas import tpu as pltpu

NUM_PROMPTS = 50
PROMPT_DIM = 768
SEQ = 577
_BB = 4


def _splice_kernel(layer_ref, x_ref, pe_ref, o_ref):
    del layer_ref
    o_ref[:, 0:1, :] = x_ref[:, 0:1, :]
    o_ref[:, 1:1 + NUM_PROMPTS, :] = jnp.broadcast_to(
        pe_ref[...], (_BB, NUM_PROMPTS, PROMPT_DIM))
    o_ref[:, 1 + NUM_PROMPTS:, :] = x_ref[:, 1:, :]


def kernel(x, prompt_embeddings, layer_num):
    Bsz = x.shape[0]
    layer = jnp.asarray(layer_num, jnp.int32).reshape((1,))
    grid_spec = pltpu.PrefetchScalarGridSpec(
        num_scalar_prefetch=1,
        grid=(Bsz // _BB,),
        in_specs=[
            pl.BlockSpec((_BB, SEQ, PROMPT_DIM), lambda b, s: (b, 0, 0)),
            pl.BlockSpec((1, NUM_PROMPTS, PROMPT_DIM),
                         lambda b, s: (s[0], 0, 0)),
        ],
        out_specs=pl.BlockSpec((_BB, SEQ + NUM_PROMPTS, PROMPT_DIM),
                               lambda b, s: (b, 0, 0)),
    )
    return pl.pallas_call(
        _splice_kernel,
        grid_spec=grid_spec,
        out_shape=jax.ShapeDtypeStruct((Bsz, SEQ + NUM_PROMPTS, PROMPT_DIM),
                                       x.dtype),
    )(layer, x, prompt_embeddings)


# speedup vs baseline: 1.0436x; 1.0027x over previous
import jax
import jax.numpy as jnp
from jax.experimental import pallas as pl
from jax.experimental.pallas import tpu as pltpu

NUM_PROMPTS = 50
PROMPT_DIM = 768
SEQ = 577
OSEQ = SEQ + NUM_PROMPTS
_D = 8


def _splice_kernel(layer_ref, x_hbm, pe_hbm, o_hbm,
                   in_buf, out_buf, pe_vmem,
                   in_sems, out_sems, stage_sem):
    bsz = x_hbm.shape[0]
    nsup = bsz // _D

    def read(b, s):
        return pltpu.make_async_copy(x_hbm.at[b], in_buf.at[s], in_sems.at[s])

    def write(s, b):
        return pltpu.make_async_copy(out_buf.at[s], o_hbm.at[b], out_sems.at[s])

    stage = pltpu.make_async_copy(pe_hbm.at[layer_ref[0]], pe_vmem, stage_sem)
    stage.start()
    for s in range(_D):
        read(s, s).start()
    stage.wait()

    def super_step(k, carry):
        for s in range(_D):
            b = k * _D + s
            read(b, s).wait()

            @pl.when(k > 0)
            def _wait_prev_write():
                write(s, b - _D).wait()

            out_buf[s, 0:1, :] = in_buf[s, 0:1, :]
            out_buf[s, 1:1 + NUM_PROMPTS, :] = pe_vmem[...]
            out_buf[s, 1 + NUM_PROMPTS:, :] = in_buf[s, 1:, :]
            write(s, b).start()

            @pl.when(k < nsup - 1)
            def _prefetch_next():
                read(b + _D, s).start()
        return carry

    jax.lax.fori_loop(0, nsup, super_step, 0)
    for s in range(_D):
        write(s, bsz - _D + s).wait()


def kernel(x, prompt_embeddings, layer_num):
    Bsz = x.shape[0]
    layer = jnp.asarray(layer_num, jnp.int32).reshape((1,))
    return pl.pallas_call(
        _splice_kernel,
        in_specs=[
            pl.BlockSpec(memory_space=pltpu.MemorySpace.SMEM),
            pl.BlockSpec(memory_space=pl.ANY),
            pl.BlockSpec(memory_space=pl.ANY),
        ],
        out_specs=pl.BlockSpec(memory_space=pl.ANY),
        out_shape=jax.ShapeDtypeStruct((Bsz, OSEQ, PROMPT_DIM), x.dtype),
        scratch_shapes=[
            pltpu.VMEM((_D, SEQ, PROMPT_DIM), jnp.float32),
            pltpu.VMEM((_D, OSEQ, PROMPT_DIM), jnp.float32),
            pltpu.VMEM((NUM_PROMPTS, PROMPT_DIM), jnp.float32),
            pltpu.SemaphoreType.DMA((_D,)),
            pltpu.SemaphoreType.DMA((_D,)),
            pltpu.SemaphoreType.DMA,
        ],
    )(layer, x, prompt_embeddings)
